# baseline (speedup 1.0000x reference)
import functools

import jax
import jax.numpy as jnp
from jax import lax
from jax.experimental import pallas as pl
from jax.experimental.pallas import tpu as pltpu
from jax.experimental.pallas import tpu_sc as plsc

_NC = 2
_NS = 16
_NW = _NC * _NS
_CHUNK = 128
_NBUF = 8


def _emb_lookup(x_LxB, weight_VxD):
    L, B = x_LxB.shape
    D = weight_VxD.shape[1]
    DT = D // 8
    NBC = B // _CHUNK
    b_per_w = B // _NW
    n_sub = b_per_w // _CHUNK
    T = L * n_sub
    assert (T - 2 * _NBUF) % _NBUF == 0
    n_groups = (T - 2 * _NBUF) // _NBUF

    mesh = plsc.VectorSubcoreMesh(core_axis_name="c", subcore_axis_name="s")

    @functools.partial(
        pl.kernel,
        mesh=mesh,
        out_type=jax.ShapeDtypeStruct((L, DT, NBC, 8, _CHUNK), jnp.float32),
        scratch_types=[
            pltpu.VMEM((L, b_per_w), jnp.int32),
            pltpu.VMEM((_NBUF, _CHUNK, D), jnp.float32),
            pltpu.VMEM((_NBUF, DT, 8, _CHUNK), jnp.float32),
            pltpu.SemaphoreType.DMA((_NBUF,)),
            pltpu.SemaphoreType.DMA((_NBUF,)),
        ],
        compiler_params=pltpu.CompilerParams(
            use_tc_tiling_on_sc=False, needs_layout_passes=False
        ),
    )
    def emb(x_hbm, table_hbm, out_hbm, idx_v, gbufs, tbufs, gsem, wsem):
        wid = lax.axis_index("s") * _NC + lax.axis_index("c")
        bbase = wid * b_per_w
        cbase = wid * n_sub

        def stage(l, c):
            pltpu.sync_copy(x_hbm.at[l, pl.ds(bbase, b_per_w)], idx_v.at[l])
            return c

        lax.fori_loop(0, L, stage, 0)

        iota16 = lax.iota(jnp.int32, 16)

        def fire_gather(j, b):
            l, s = j // n_sub, j % n_sub
            pltpu.make_async_copy(
                table_hbm.at[idx_v.at[l, pl.ds(s * _CHUNK, _CHUNK)]],
                gbufs.at[b], gsem.at[b],
            ).start()

        def wait_gather(j, b):
            l, s = j // n_sub, j % n_sub
            pltpu.make_async_copy(
                table_hbm.at[idx_v.at[l, pl.ds(s * _CHUNK, _CHUNK)]],
                gbufs.at[b], gsem.at[b],
            ).wait()

        def shuffle(b, m):
            def block(rb, c):
                r0 = rb * 16
                rows = r0 + iota16
                for d in range(D):
                    v = plsc.load_gather(
                        gbufs.at[b], [rows, jnp.full((16,), d, jnp.int32)]
                    )
                    tbufs[m, d // 8, d % 8, pl.ds(r0, 16)] = v
                return c

            lax.fori_loop(0, _CHUNK // 16, block, 0)

        def fire_wb(j, m):
            l, s = j // n_sub, j % n_sub
            pltpu.make_async_copy(
                tbufs.at[m],
                out_hbm.at[l, :, cbase + s],
                wsem.at[m],
            ).start()

        def wait_wb(j, m):
            l, s = j // n_sub, j % n_sub
            pltpu.make_async_copy(
                tbufs.at[m],
                out_hbm.at[l, :, cbase + s],
                wsem.at[m],
            ).wait()

        for c in range(_NBUF):
            fire_gather(c, c)
        for j in range(_NBUF):
            wait_gather(j, j)
            shuffle(j, j)
            fire_wb(j, j)
            fire_gather(j + _NBUF, j)

        def group(g, carry):
            for b in range(_NBUF):
                j = _NBUF + g * _NBUF + b
                wait_gather(j, b)
                wait_wb(j - _NBUF, b)
                shuffle(b, b)
                fire_wb(j, b)
                fire_gather(j + _NBUF, b)
            return carry

        lax.fori_loop(0, n_groups, group, 0)

        for t in range(_NBUF):
            j = T - _NBUF + t
            b = j % _NBUF
            wait_gather(j, b)
            wait_wb(j - _NBUF, b)
            shuffle(b, b)
            fire_wb(j, b)
        for t in range(_NBUF):
            j = T - _NBUF + t
            wait_wb(j, j % _NBUF)

    return emb(x_LxB, weight_VxD)


def kernel(x_T, weight_VxD):
    B, L = x_T.shape
    V, D = weight_VxD.shape
    x_LxB = x_T.T.astype(jnp.int32)
    out_t = _emb_lookup(x_LxB, weight_VxD)
    return out_t.transpose(2, 4, 0, 1, 3).reshape(B, L, D)


# speedup vs baseline: 1.7113x; 1.7113x over previous
import functools

import jax
import jax.numpy as jnp
from jax import lax
from jax.experimental import pallas as pl
from jax.experimental.pallas import tpu as pltpu
from jax.experimental.pallas import tpu_sc as plsc

_NC = 2
_NS = 16
_NW = _NC * _NS
_CHUNK = 128
_NBUF = 8


def _emb_lookup(x_LxB, weight_VxD):
    L, B = x_LxB.shape
    D = weight_VxD.shape[1]
    DT = D // 8
    NBC = B // _CHUNK
    b_per_w = B // _NW
    n_sub = b_per_w // _CHUNK
    T = L * n_sub
    assert (T - 2 * _NBUF) % _NBUF == 0
    n_groups = (T - 2 * _NBUF) // _NBUF

    mesh = plsc.VectorSubcoreMesh(core_axis_name="c", subcore_axis_name="s")

    @functools.partial(
        pl.kernel,
        mesh=mesh,
        out_type=jax.ShapeDtypeStruct((L, DT, NBC, 8, _CHUNK), jnp.float32),
        scratch_types=[
            pltpu.VMEM((L, b_per_w), jnp.int32),
            pltpu.VMEM((_NBUF, _CHUNK, D), jnp.float32),
            pltpu.VMEM((_NBUF, DT, 8, _CHUNK + 1), jnp.float32),
            pltpu.SemaphoreType.DMA((_NBUF,)),
            pltpu.SemaphoreType.DMA((_NBUF,)),
        ],
        compiler_params=pltpu.CompilerParams(
            use_tc_tiling_on_sc=False, needs_layout_passes=False
        ),
    )
    def emb(x_hbm, table_hbm, out_hbm, idx_v, gbufs, tbufs, gsem, wsem):
        wid = lax.axis_index("s") * _NC + lax.axis_index("c")
        bbase = wid * b_per_w
        cbase = wid * n_sub

        def stage(l, c):
            pltpu.sync_copy(x_hbm.at[l, pl.ds(bbase, b_per_w)], idx_v.at[l])
            return c

        lax.fori_loop(0, L, stage, 0)

        iota16 = lax.iota(jnp.int32, 16)
        tr_lo, rd_lo = iota16 // 8, iota16 % 8
        tr_hi, rd_hi = tr_lo + 2, rd_lo

        def fire_gather(j, b):
            l, s = j // n_sub, j % n_sub
            pltpu.make_async_copy(
                table_hbm.at[idx_v.at[l, pl.ds(s * _CHUNK, _CHUNK)]],
                gbufs.at[b], gsem.at[b],
            ).start()

        def wait_gather(j, b):
            l, s = j // n_sub, j % n_sub
            pltpu.make_async_copy(
                table_hbm.at[idx_v.at[l, pl.ds(s * _CHUNK, _CHUNK)]],
                gbufs.at[b], gsem.at[b],
            ).wait()

        def shuffle(b, m):
            def rows8(r8, c):
                r0 = r8 * 8
                for i in range(8):
                    r = r0 + i
                    v0 = gbufs[b, r, pl.ds(0, 16)]
                    v1 = gbufs[b, r, pl.ds(16, 16)]
                    rv = jnp.full((16,), r, jnp.int32)
                    plsc.store_scatter(tbufs.at[m], [tr_lo, rd_lo, rv], v0)
                    plsc.store_scatter(tbufs.at[m], [tr_hi, rd_hi, rv], v1)
                return c

            lax.fori_loop(0, _CHUNK // 8, rows8, 0)

        def fire_wb(j, m):
            l, s = j // n_sub, j % n_sub
            pltpu.make_async_copy(
                tbufs.at[m, :, :, pl.ds(0, _CHUNK)],
                out_hbm.at[l, :, cbase + s],
                wsem.at[m],
            ).start()

        def wait_wb(j, m):
            l, s = j // n_sub, j % n_sub
            pltpu.make_async_copy(
                tbufs.at[m, :, :, pl.ds(0, _CHUNK)],
                out_hbm.at[l, :, cbase + s],
                wsem.at[m],
            ).wait()

        for c in range(_NBUF):
            fire_gather(c, c)
        for j in range(_NBUF):
            wait_gather(j, j)
            shuffle(j, j)
            fire_wb(j, j)
            fire_gather(j + _NBUF, j)

        def group(g, carry):
            for b in range(_NBUF):
                j = _NBUF + g * _NBUF + b
                wait_gather(j, b)
                wait_wb(j - _NBUF, b)
                shuffle(b, b)
                fire_wb(j, b)
                fire_gather(j + _NBUF, b)
            return carry

        lax.fori_loop(0, n_groups, group, 0)

        for t in range(_NBUF):
            j = T - _NBUF + t
            b = j % _NBUF
            wait_gather(j, b)
            wait_wb(j - _NBUF, b)
            shuffle(b, b)
            fire_wb(j, b)
        for t in range(_NBUF):
            j = T - _NBUF + t
            wait_wb(j, j % _NBUF)

    return emb(x_LxB, weight_VxD)


def kernel(x_T, weight_VxD):
    B, L = x_T.shape
    V, D = weight_VxD.shape
    x_LxB = x_T.T.astype(jnp.int32)
    out_t = _emb_lookup(x_LxB, weight_VxD)
    return out_t.transpose(2, 4, 0, 1, 3).reshape(B, L, D)
